# baseline (speedup 1.0000x reference)
import functools

import jax
import jax.numpy as jnp
from jax import lax
from jax.experimental import pallas as pl
from jax.experimental.pallas import tpu as pltpu
from jax.experimental.pallas import tpu_sc as plsc

K = 512
VPAD = 100352
CHUNK = 1024
CAP = 2048
NB = 1024
NC, NS, L = 2, 16, 16
NW = NC * NS

HIST_LO = -3.0e-5
HIST_HI = 0.0310
BIN_W = (HIST_HI - HIST_LO) / NB


def _dist_body(sw_ref, sw2_ref, w_ref, w2_ref, out_ref):
    sw = sw_ref[...]
    wc = w_ref[...]
    ww = 2.0 * lax.dot_general(
        sw, wc, (((1,), (1,)), ((), ())),
        preferred_element_type=jnp.float32)
    sw2 = sw2_ref[...]
    w2 = w2_ref[...]
    out_ref[...] = (sw2 - ww) + w2


def _distances(sample_w, sample_w2, wpad, w2pad):
    B = sample_w.shape[0]
    grid = VPAD // CHUNK
    return pl.pallas_call(
        _dist_body,
        grid=(grid,),
        in_specs=[
            pl.BlockSpec((B, 128), lambda c: (0, 0)),
            pl.BlockSpec((B, 1), lambda c: (0, 0)),
            pl.BlockSpec((CHUNK, 128), lambda c: (c, 0)),
            pl.BlockSpec((1, CHUNK), lambda c: (0, c)),
        ],
        out_specs=pl.BlockSpec((B, CHUNK), lambda c: (0, c)),
        out_shape=jax.ShapeDtypeStruct((B, VPAD), jnp.float32),
    )(sample_w, sample_w2, wpad, w2pad)


def _sc_select(dis):
    B = dis.shape[0]
    rows_per_w = B // NW
    nvec = VPAD // L

    mesh = plsc.VectorSubcoreMesh(core_axis_name="c", subcore_axis_name="s")

    @functools.partial(
        pl.kernel,
        mesh=mesh,
        out_type=[
            jax.ShapeDtypeStruct((B, CAP), jnp.float32),
            jax.ShapeDtypeStruct((B, CAP), jnp.int32),
            jax.ShapeDtypeStruct((B, L), jnp.int32),
        ],
        scratch_types=[
            pltpu.VMEM((VPAD,), jnp.float32),
            pltpu.VMEM((L * NB,), jnp.int32),
            pltpu.VMEM((CAP,), jnp.float32),
            pltpu.VMEM((CAP,), jnp.int32),
            pltpu.VMEM((L,), jnp.int32),
        ],
        compiler_params=pltpu.CompilerParams(needs_layout_passes=False),
    )
    def select(dis_hbm, cvals_hbm, cidx_hbm, cnt_hbm,
               row_v, hist_v, cvals_v, cidx_v, cnt_v):
        wid = lax.axis_index("s") * NC + lax.axis_index("c")
        lane = lax.broadcasted_iota(jnp.int32, (L,), 0)
        ones = jnp.ones((L,), jnp.int32)
        zeros = jnp.zeros((L,), jnp.int32)
        inf16 = jnp.full((L,), jnp.inf, jnp.float32)

        def per_row(r_local, _):
            row = wid * rows_per_w + r_local
            pltpu.sync_copy(dis_hbm.at[row], row_v)

            def zero_hist(j, _):
                hist_v[pl.ds(j * L, L)] = zeros
                return 0
            lax.fori_loop(0, (L * NB) // L, zero_hist, 0)

            def hist_body(i, _):
                v = row_v[pl.ds(i * (4 * L), L)]
                v = jnp.minimum(jnp.maximum(v, HIST_LO),
                                HIST_LO + (NB - 0.5) * BIN_W)
                b = ((v - HIST_LO) * (1.0 / BIN_W)).astype(jnp.int32)
                plsc.addupdate_scatter(hist_v, [lane * NB + b], ones)
                return 0
            lax.fori_loop(0, nvec // 4, hist_body, 0)

            def cum_body(c, carry):
                cum_carry, nbins_ge = carry
                acc = hist_v[pl.ds(c * L, L)]
                for l in range(1, L):
                    acc = acc + hist_v[pl.ds(l * NB + c * L, L)]
                cum = cum_carry + plsc.cumsum(acc)
                ge = plsc.all_reduce_population_count(cum >= 220)
                cum_carry = cum_carry + jnp.broadcast_to(jnp.sum(acc), (L,))
                return (cum_carry, nbins_ge + ge)
            _, nbins_ge = lax.fori_loop(
                0, NB // L, cum_body,
                (jnp.zeros((L,), jnp.int32), jnp.zeros((L,), jnp.int32)))
            bin_idx = NB - jnp.max(nbins_ge)
            thr = HIST_LO + (bin_idx.astype(jnp.float32) + 2.0) * BIN_W

            def prefill(j, _):
                cvals_v[pl.ds(j * L, L)] = inf16
                return 0
            lax.fori_loop(0, CAP // L, prefill, 0)

            def compact_body(i, base):
                v = row_v[pl.ds(i * L, L)]
                m = v <= thr
                cnt = plsc.all_reduce_population_count(m)

                def do_store(base):
                    mi = jnp.where(m, 1, 0)
                    offs = base + plsc.cumsum(mi) - 1
                    okm = jnp.logical_and(m, offs < CAP)
                    plsc.store_scatter(cvals_v, [offs], v, mask=okm)
                    plsc.store_scatter(cidx_v, [offs], lane + i * L,
                                       mask=okm)
                    return base + cnt

                return lax.cond(jnp.max(cnt) > 0, do_store, lambda b: b,
                                base)
            count = lax.fori_loop(0, nvec, compact_body, zeros)

            cnt_v[...] = count
            pltpu.sync_copy(cvals_v, cvals_hbm.at[row])
            pltpu.sync_copy(cidx_v, cidx_hbm.at[row])
            pltpu.sync_copy(cnt_v, cnt_hbm.at[row])
            return 0

        lax.fori_loop(0, rows_per_w, per_row, 0)

    return select(dis)


def kernel(x, mu, var, labels, weight, bias):
    B, d = x.shape
    V = weight.shape[0]
    wpad = jnp.pad(weight, ((0, VPAD - V), (0, 0)))
    sample_weight = jnp.take(weight, labels, axis=0)
    sample_w2 = jnp.sum(sample_weight ** 2, axis=1, keepdims=True)
    w2 = jnp.sum(weight ** 2, axis=1)
    w2pad = jnp.pad(w2, (0, VPAD - V), constant_values=jnp.inf)[None, :]

    dis = _distances(sample_weight, sample_w2, wpad, w2pad)
    cvals, cidx, counts = _sc_select(dis)

    cnt = counts[:, 0]
    ok = jnp.logical_and(jnp.all(cnt >= K), jnp.all(cnt <= CAP))

    def _fast(ops):
        cvals, cidx, _ = ops
        _, pos = lax.top_k(-cvals, K)
        return jnp.take_along_axis(cidx, pos, axis=1)

    def _slow(ops):
        _, _, dis = ops
        _, idx = lax.top_k(-dis, K)
        return idx

    topk_indice = lax.cond(ok, _fast, _slow, (cvals, cidx, dis))

    topk_weight = jnp.take(weight, topk_indice, axis=0)
    topk_bias = jnp.take(bias, topk_indice, axis=0)
    all_class_density = jnp.exp(-((topk_weight - mu[:, None, :]) ** 2)
                                / (2.0 * var[:, None, :]))
    confid = all_class_density / jnp.clip(
        jnp.sum(all_class_density, axis=1, keepdims=True), 1e-08, None)
    max_confid = jnp.max(confid, axis=1, keepdims=True)
    nontrivial = (confid >= jnp.clip(max_confid * 0.5, None, 0.1))
    masked = topk_weight * nontrivial.astype(topk_weight.dtype)
    score = jnp.squeeze(
        jnp.matmul(x[:, None, :], jnp.transpose(masked, (0, 2, 1))),
        axis=1) + topk_bias
    return (score, topk_indice, all_class_density, nontrivial)


# speedup vs baseline: 1.2988x; 1.2988x over previous
import functools

import jax
import jax.numpy as jnp
from jax import lax
from jax.experimental import pallas as pl
from jax.experimental.pallas import tpu as pltpu
from jax.experimental.pallas import tpu_sc as plsc

K = 512
VPAD = 100352
CHUNK = 1024
CAP = 2048
NB = 1024
NC, NS, L = 2, 16, 16
NW = NC * NS

HIST_LO = -3.0e-5
HIST_HI = 0.0310
BIN_W = (HIST_HI - HIST_LO) / NB


def _dist_body(sw_ref, sw2_ref, w_ref, w2_ref, out_ref):
    sw = sw_ref[...]
    wc = w_ref[...]
    ww = 2.0 * lax.dot_general(
        sw, wc, (((1,), (1,)), ((), ())),
        preferred_element_type=jnp.float32)
    sw2 = sw2_ref[...]
    w2 = w2_ref[...]
    out_ref[...] = (sw2 - ww) + w2


def _distances(sample_w, sample_w2, wpad, w2pad):
    B = sample_w.shape[0]
    grid = VPAD // CHUNK
    return pl.pallas_call(
        _dist_body,
        grid=(grid,),
        in_specs=[
            pl.BlockSpec((B, 128), lambda c: (0, 0)),
            pl.BlockSpec((B, 1), lambda c: (0, 0)),
            pl.BlockSpec((CHUNK, 128), lambda c: (c, 0)),
            pl.BlockSpec((1, CHUNK), lambda c: (0, c)),
        ],
        out_specs=pl.BlockSpec((B, CHUNK), lambda c: (0, c)),
        out_shape=jax.ShapeDtypeStruct((B, VPAD), jnp.float32),
    )(sample_w, sample_w2, wpad, w2pad)


def _sc_select(dis):
    B = dis.shape[0]
    rows_per_w = B // NW
    nvec = VPAD // L

    mesh = plsc.VectorSubcoreMesh(core_axis_name="c", subcore_axis_name="s")

    @functools.partial(
        pl.kernel,
        mesh=mesh,
        out_type=[
            jax.ShapeDtypeStruct((B, CAP), jnp.float32),
            jax.ShapeDtypeStruct((B, CAP), jnp.int32),
            jax.ShapeDtypeStruct((B, L), jnp.int32),
        ],
        scratch_types=[
            pltpu.VMEM((VPAD,), jnp.float32),
            pltpu.VMEM((L * NB,), jnp.int32),
            pltpu.VMEM((CAP,), jnp.float32),
            pltpu.VMEM((CAP,), jnp.int32),
            pltpu.VMEM((L,), jnp.int32),
        ],
        compiler_params=pltpu.CompilerParams(needs_layout_passes=False),
    )
    def select(dis_hbm, cvals_hbm, cidx_hbm, cnt_hbm,
               row_v, hist_v, cvals_v, cidx_v, cnt_v):
        wid = lax.axis_index("s") * NC + lax.axis_index("c")
        lane = lax.broadcasted_iota(jnp.int32, (L,), 0)
        ones = jnp.ones((L,), jnp.int32)
        zeros = jnp.zeros((L,), jnp.int32)
        inf16 = jnp.full((L,), jnp.inf, jnp.float32)

        def per_row(r_local, _):
            row = wid * rows_per_w + r_local
            pltpu.sync_copy(dis_hbm.at[row], row_v)

            def zero_hist(j, _):
                hist_v[pl.ds(j * L, L)] = zeros
                return 0
            lax.fori_loop(0, (L * NB) // L, zero_hist, 0)

            def hist_body(i, _):
                v = row_v[pl.ds(i * (4 * L), L)]
                v = jnp.minimum(jnp.maximum(v, HIST_LO),
                                HIST_LO + (NB - 0.5) * BIN_W)
                b = ((v - HIST_LO) * (1.0 / BIN_W)).astype(jnp.int32)
                plsc.addupdate_scatter(hist_v, [lane * NB + b], ones)
                return 0
            lax.fori_loop(0, nvec // 4, hist_body, 0)

            def cum_body(c, carry):
                cum_carry, nbins_ge = carry
                acc = hist_v[pl.ds(c * L, L)]
                for l in range(1, L):
                    acc = acc + hist_v[pl.ds(l * NB + c * L, L)]
                cum = cum_carry + plsc.cumsum(acc)
                ge = plsc.all_reduce_population_count(cum >= 220)
                cum_carry = cum_carry + jnp.broadcast_to(jnp.sum(acc), (L,))
                return (cum_carry, nbins_ge + ge)
            _, nbins_ge = lax.fori_loop(
                0, NB // L, cum_body,
                (jnp.zeros((L,), jnp.int32), jnp.zeros((L,), jnp.int32)))
            bin_idx = NB - jnp.max(nbins_ge)
            thr = HIST_LO + (bin_idx.astype(jnp.float32) + 2.0) * BIN_W

            def prefill(j, _):
                cvals_v[pl.ds(j * L, L)] = inf16
                return 0
            lax.fori_loop(0, CAP // L, prefill, 0)

            def compact_body(i, base):
                v = row_v[pl.ds(i * L, L)]
                m = v <= thr
                mi = jnp.where(m, 1, 0)
                offs = base + plsc.cumsum(mi) - 1
                okm = jnp.logical_and(m, offs < CAP)
                plsc.store_scatter(cvals_v, [offs], v, mask=okm)
                plsc.store_scatter(cidx_v, [offs], lane + i * L, mask=okm)
                return base + plsc.all_reduce_population_count(m)
            count = lax.fori_loop(0, nvec, compact_body, zeros)

            cnt_v[...] = count
            pltpu.sync_copy(cvals_v, cvals_hbm.at[row])
            pltpu.sync_copy(cidx_v, cidx_hbm.at[row])
            pltpu.sync_copy(cnt_v, cnt_hbm.at[row])
            return 0

        lax.fori_loop(0, rows_per_w, per_row, 0)

    return select(dis)


def _tail_body(w_ref, mu_ref, var_ref, x_ref, dens_ref, nt_ref, score_ref):
    w = w_ref[...]
    mu = mu_ref[...][:, None, :]
    var = var_ref[...][:, None, :]
    x = x_ref[...][:, None, :]
    dens = jnp.exp(-((w - mu) ** 2) / (2.0 * var))
    dens_ref[...] = dens
    s = jnp.clip(jnp.sum(dens, axis=1, keepdims=True), 1e-08, None)
    confid = dens / s
    maxc = jnp.max(confid, axis=1, keepdims=True)
    nt = (confid >= jnp.clip(maxc * 0.5, None, 0.1)).astype(jnp.float32)
    nt_ref[...] = nt
    score_ref[...] = jnp.sum(w * nt * x, axis=2)


def _tail(topk_weight, mu, var, x):
    B, _, d = topk_weight.shape
    RB = 8
    grid = B // RB
    return pl.pallas_call(
        _tail_body,
        grid=(grid,),
        in_specs=[
            pl.BlockSpec((RB, K, d), lambda r: (r, 0, 0)),
            pl.BlockSpec((RB, d), lambda r: (r, 0)),
            pl.BlockSpec((RB, d), lambda r: (r, 0)),
            pl.BlockSpec((RB, d), lambda r: (r, 0)),
        ],
        out_specs=[
            pl.BlockSpec((RB, K, d), lambda r: (r, 0, 0)),
            pl.BlockSpec((RB, K, d), lambda r: (r, 0, 0)),
            pl.BlockSpec((RB, K), lambda r: (r, 0)),
        ],
        out_shape=[
            jax.ShapeDtypeStruct((B, K, d), jnp.float32),
            jax.ShapeDtypeStruct((B, K, d), jnp.float32),
            jax.ShapeDtypeStruct((B, K), jnp.float32),
        ],
    )(topk_weight, mu, var, x)


def kernel(x, mu, var, labels, weight, bias):
    B, d = x.shape
    V = weight.shape[0]
    wpad = jnp.pad(weight, ((0, VPAD - V), (0, 0)))
    sample_weight = jnp.take(weight, labels, axis=0)
    sample_w2 = jnp.sum(sample_weight ** 2, axis=1, keepdims=True)
    w2 = jnp.sum(weight ** 2, axis=1)
    w2pad = jnp.pad(w2, (0, VPAD - V), constant_values=jnp.inf)[None, :]

    dis = _distances(sample_weight, sample_w2, wpad, w2pad)
    cvals, cidx, counts = _sc_select(dis)

    cnt = counts[:, 0]
    ok = jnp.logical_and(jnp.all(cnt >= K), jnp.all(cnt <= CAP))

    def _fast(ops):
        cvals, cidx, _ = ops
        _, pos = lax.top_k(-cvals, K)
        return jnp.take_along_axis(cidx, pos, axis=1)

    def _slow(ops):
        _, _, dis = ops
        _, idx = lax.top_k(-dis, K)
        return idx

    topk_indice = lax.cond(ok, _fast, _slow, (cvals, cidx, dis))

    topk_weight = jnp.take(weight, topk_indice, axis=0)
    topk_bias = jnp.take(bias, topk_indice, axis=0)
    all_class_density, nt_f32, score0 = _tail(topk_weight, mu, var, x)
    nontrivial = nt_f32 > 0.0
    score = score0 + topk_bias
    return (score, topk_indice, all_class_density, nontrivial)


# speedup vs baseline: 1.5717x; 1.2102x over previous
import functools

import jax
import jax.numpy as jnp
from jax import lax
from jax.experimental import pallas as pl
from jax.experimental.pallas import tpu as pltpu
from jax.experimental.pallas import tpu_sc as plsc

K = 512
VPAD = 100352
CHUNK = 1024
CAP = 2048
NB = 1024
NC, NS, L = 2, 16, 16
NW = NC * NS

HIST_LO = -3.0e-5
HIST_HI = 0.0310
BIN_W = (HIST_HI - HIST_LO) / NB


def _dist_body(sw_ref, sw2_ref, w_ref, w2_ref, out_ref):
    sw = sw_ref[...]
    wc = w_ref[...]
    ww = 2.0 * lax.dot_general(
        sw, wc, (((1,), (1,)), ((), ())),
        preferred_element_type=jnp.float32)
    sw2 = sw2_ref[...]
    w2 = w2_ref[...]
    out_ref[...] = (sw2 - ww) + w2


def _distances(sample_w, sample_w2, wpad, w2pad):
    B = sample_w.shape[0]
    grid = VPAD // CHUNK
    return pl.pallas_call(
        _dist_body,
        grid=(grid,),
        in_specs=[
            pl.BlockSpec((B, 128), lambda c: (0, 0)),
            pl.BlockSpec((B, 1), lambda c: (0, 0)),
            pl.BlockSpec((CHUNK, 128), lambda c: (c, 0)),
            pl.BlockSpec((1, CHUNK), lambda c: (0, c)),
        ],
        out_specs=pl.BlockSpec((B, CHUNK), lambda c: (0, c)),
        out_shape=jax.ShapeDtypeStruct((B, VPAD), jnp.float32),
    )(sample_w, sample_w2, wpad, w2pad)


def _sc_select(dis):
    B = dis.shape[0]
    rows_per_w = B // NW
    nvec = VPAD // L

    mesh = plsc.VectorSubcoreMesh(core_axis_name="c", subcore_axis_name="s")

    @functools.partial(
        pl.kernel,
        mesh=mesh,
        out_type=[
            jax.ShapeDtypeStruct((B, CAP), jnp.float32),
            jax.ShapeDtypeStruct((B, CAP), jnp.int32),
            jax.ShapeDtypeStruct((B, L), jnp.int32),
        ],
        scratch_types=[
            pltpu.VMEM((VPAD,), jnp.float32),
            pltpu.VMEM((L * NB,), jnp.int32),
            pltpu.VMEM((CAP,), jnp.float32),
            pltpu.VMEM((CAP,), jnp.int32),
            pltpu.VMEM((L,), jnp.int32),
        ],
        compiler_params=pltpu.CompilerParams(needs_layout_passes=False),
    )
    def select(dis_hbm, cvals_hbm, cidx_hbm, cnt_hbm,
               row_v, hist_v, cvals_v, cidx_v, cnt_v):
        wid = lax.axis_index("s") * NC + lax.axis_index("c")
        lane = lax.broadcasted_iota(jnp.int32, (L,), 0)
        ones = jnp.ones((L,), jnp.int32)
        zeros = jnp.zeros((L,), jnp.int32)
        inf16 = jnp.full((L,), jnp.inf, jnp.float32)

        def per_row(r_local, _):
            row = wid * rows_per_w + r_local
            pltpu.sync_copy(dis_hbm.at[row], row_v)

            def zero_hist(j, _):
                hist_v[pl.ds(j * L, L)] = zeros
                return 0
            lax.fori_loop(0, (L * NB) // L, zero_hist, 0)

            def hist_body(i, _):
                v = row_v[pl.ds(i * (4 * L), L)]
                v = jnp.minimum(jnp.maximum(v, HIST_LO),
                                HIST_LO + (NB - 0.5) * BIN_W)
                b = ((v - HIST_LO) * (1.0 / BIN_W)).astype(jnp.int32)
                plsc.addupdate_scatter(hist_v, [lane * NB + b], ones)
                return 0
            lax.fori_loop(0, nvec // 4, hist_body, 0)

            def cum_body(c, carry):
                cum_carry, nbins_ge = carry
                acc = hist_v[pl.ds(c * L, L)]
                for l in range(1, L):
                    acc = acc + hist_v[pl.ds(l * NB + c * L, L)]
                cum = cum_carry + plsc.cumsum(acc)
                ge = plsc.all_reduce_population_count(cum >= 220)
                cum_carry = cum_carry + jnp.broadcast_to(jnp.sum(acc), (L,))
                return (cum_carry, nbins_ge + ge)
            _, nbins_ge = lax.fori_loop(
                0, NB // L, cum_body,
                (jnp.zeros((L,), jnp.int32), jnp.zeros((L,), jnp.int32)))
            bin_idx = NB - jnp.max(nbins_ge)
            thr = HIST_LO + (bin_idx.astype(jnp.float32) + 2.0) * BIN_W

            def prefill(j, _):
                cvals_v[pl.ds(j * L, L)] = inf16
                return 0
            lax.fori_loop(0, CAP // L, prefill, 0)

            def compact_body(i, base):
                v = row_v[pl.ds(i * L, L)]
                m = v <= thr
                mi = jnp.where(m, 1, 0)
                offs = base + plsc.cumsum(mi) - 1
                okm = jnp.logical_and(m, offs < CAP)
                plsc.store_scatter(cvals_v, [offs], v, mask=okm)
                plsc.store_scatter(cidx_v, [offs], lane + i * L, mask=okm)
                return base + plsc.all_reduce_population_count(m)
            count = lax.fori_loop(0, nvec, compact_body, zeros)

            cnt_v[...] = count
            pltpu.sync_copy(cvals_v, cvals_hbm.at[row])
            pltpu.sync_copy(cidx_v, cidx_hbm.at[row])
            pltpu.sync_copy(cnt_v, cnt_hbm.at[row])
            return 0

        lax.fori_loop(0, rows_per_w, per_row, 0)

    return select(dis)


def _sc_gather(table, idx_flat):
    NR = idx_flat.shape[0]
    D = table.shape[1]
    per_w = NR // NW
    CH = 128
    nch = per_w // CH

    mesh = plsc.VectorSubcoreMesh(core_axis_name="c", subcore_axis_name="s")

    @functools.partial(
        pl.kernel,
        mesh=mesh,
        out_type=jax.ShapeDtypeStruct((NR, D), jnp.float32),
        scratch_types=[
            pltpu.VMEM((CH,), jnp.int32),
            pltpu.VMEM((CH, D), jnp.float32),
            pltpu.SemaphoreType.DMA,
        ],
        compiler_params=pltpu.CompilerParams(needs_layout_passes=False),
    )
    def g(table_hbm, idx_hbm, out_hbm, idx_v, rows_v, sem):
        wid = lax.axis_index("s") * NC + lax.axis_index("c")

        def chunk(i, _):
            base = wid * per_w + i * CH
            pltpu.sync_copy(idx_hbm.at[pl.ds(base, CH)], idx_v)
            pltpu.async_copy(table_hbm.at[idx_v], rows_v, sem).wait()
            pltpu.sync_copy(rows_v, out_hbm.at[pl.ds(base, CH)])
            return 0

        lax.fori_loop(0, nch, chunk, 0)

    return g(table, idx_flat)


def _tail_body(w_ref, mu_ref, var_ref, x_ref, dens_ref, nt_ref, score_ref):
    w = w_ref[...]
    mu = mu_ref[...][:, None, :]
    var = var_ref[...][:, None, :]
    x = x_ref[...][:, None, :]
    dens = jnp.exp(-((w - mu) ** 2) / (2.0 * var))
    dens_ref[...] = dens
    s = jnp.clip(jnp.sum(dens, axis=1, keepdims=True), 1e-08, None)
    confid = dens / s
    maxc = jnp.max(confid, axis=1, keepdims=True)
    nt = (confid >= jnp.clip(maxc * 0.5, None, 0.1)).astype(jnp.float32)
    nt_ref[...] = nt
    score_ref[...] = jnp.sum(w * nt * x, axis=2)


def _tail(topk_weight, mu, var, x):
    B, _, d = topk_weight.shape
    RB = 8
    grid = B // RB
    return pl.pallas_call(
        _tail_body,
        grid=(grid,),
        in_specs=[
            pl.BlockSpec((RB, K, d), lambda r: (r, 0, 0)),
            pl.BlockSpec((RB, d), lambda r: (r, 0)),
            pl.BlockSpec((RB, d), lambda r: (r, 0)),
            pl.BlockSpec((RB, d), lambda r: (r, 0)),
        ],
        out_specs=[
            pl.BlockSpec((RB, K, d), lambda r: (r, 0, 0)),
            pl.BlockSpec((RB, K, d), lambda r: (r, 0, 0)),
            pl.BlockSpec((RB, K), lambda r: (r, 0)),
        ],
        out_shape=[
            jax.ShapeDtypeStruct((B, K, d), jnp.float32),
            jax.ShapeDtypeStruct((B, K, d), jnp.float32),
            jax.ShapeDtypeStruct((B, K), jnp.float32),
        ],
    )(topk_weight, mu, var, x)


def kernel(x, mu, var, labels, weight, bias):
    B, d = x.shape
    V = weight.shape[0]
    wpad = jnp.pad(weight, ((0, VPAD - V), (0, 0)))
    sample_weight = jnp.take(weight, labels, axis=0)
    sample_w2 = jnp.sum(sample_weight ** 2, axis=1, keepdims=True)
    w2 = jnp.sum(weight ** 2, axis=1)
    w2pad = jnp.pad(w2, (0, VPAD - V), constant_values=jnp.inf)[None, :]

    dis = _distances(sample_weight, sample_w2, wpad, w2pad)
    cvals, cidx, counts = _sc_select(dis)

    cnt = counts[:, 0]
    ok = jnp.logical_and(jnp.all(cnt >= K), jnp.all(cnt <= CAP))

    def _fast(ops):
        cvals, cidx, _ = ops
        _, pos = lax.top_k(-cvals, K)
        return jnp.take_along_axis(cidx, pos, axis=1)

    def _slow(ops):
        _, _, dis = ops
        _, idx = lax.top_k(-dis, K)
        return idx

    topk_indice = lax.cond(ok, _fast, _slow, (cvals, cidx, dis))

    topk_weight = _sc_gather(wpad, topk_indice.reshape(-1)).reshape(B, K, d)
    topk_bias = jnp.take(bias, topk_indice, axis=0)
    all_class_density, nt_f32, score0 = _tail(topk_weight, mu, var, x)
    nontrivial = nt_f32 > 0.0
    score = score0 + topk_bias
    return (score, topk_indice, all_class_density, nontrivial)
